# Initial kernel scaffold; baseline (speedup 1.0000x reference)
import jax
import jax.numpy as jnp
from jax.experimental import pallas as pl


def kernel(input, target):
    raise NotImplementedError("write your pallas kernel here")


# speedup vs baseline: 249.0463x; 249.0463x over previous
import functools

import jax
import jax.numpy as jnp
from jax import lax
from jax.experimental import pallas as pl
from jax.experimental.pallas import tpu as pltpu
from jax.experimental.pallas import tpu_sc as plsc

_ALPHA = 1.0
_IGNORE = 0
_LANES = 16


def _first_occurrence(t, n, v):
    mesh = plsc.VectorSubcoreMesh(core_axis_name="c", subcore_axis_name="s")

    @functools.partial(
        pl.kernel,
        mesh=mesh,
        out_type=jax.ShapeDtypeStruct((v,), jnp.int32),
        scratch_types=[
            pltpu.VMEM((n,), jnp.int32),
            pltpu.VMEM((v,), jnp.int32),
        ],
        compiler_params=pltpu.CompilerParams(needs_layout_passes=False),
    )
    def body(t_hbm, out_hbm, t_vmem, first_vmem):
        cid = lax.axis_index("c")
        sid = lax.axis_index("s")

        @pl.when(jnp.logical_and(cid == 0, sid == 0))
        def _():
            pltpu.sync_copy(t_hbm, t_vmem)
            fill = jnp.full((_LANES,), n, jnp.int32)

            def init(k, carry):
                first_vmem[pl.ds(k * _LANES, _LANES)] = fill
                return carry

            lax.fori_loop(0, v // _LANES, init, 0)

            n_chunks = n // _LANES
            lanes = lax.broadcasted_iota(jnp.int32, (_LANES,), 0)

            def chunk(c, carry):
                base = (n_chunks - 1 - c) * _LANES
                tj = t_vmem[pl.ds(base, _LANES)]
                jv = lanes + base
                for l in range(_LANES - 1, -1, -1):
                    plsc.store_scatter(first_vmem, [tj], jv, mask=lanes == l)
                return carry

            lax.fori_loop(0, n_chunks, chunk, 0)
            pltpu.sync_copy(first_vmem, out_hbm)

    return body(t)


def _loss_body(x_ref, t_ref, first_ref, loss_ref, valid_ref, *, rows_per_blk):
    i = pl.program_id(0)
    x = x_ref[...]
    t = t_ref[...]
    first = first_ref[...]

    m = jnp.max(x, axis=1, keepdims=True)
    e = jnp.exp(x - m)
    s = jnp.sum(e, axis=1, keepdims=True)
    lse = m + jnp.log(s)

    rows = i * rows_per_blk + lax.broadcasted_iota(
        jnp.int32, (rows_per_blk, 1), 0)
    viota = lax.broadcasted_iota(jnp.int32, (1, x.shape[1]), 1)

    p = e * (1.0 / s)
    f = -jnp.log(jnp.maximum(1.0 - p, 1e-5))
    neg_mask = (first < rows) & (viota != _IGNORE) & (viota != t)
    ul = jnp.sum(jnp.where(neg_mask, f, 0.0), axis=1, keepdims=True)

    onehot = viota == t
    xt = jnp.sum(jnp.where(onehot, x, 0.0), axis=1, keepdims=True)
    nll = lse - xt

    valid = t != _IGNORE
    loss_ref[...] = jnp.where(valid, _ALPHA * ul + nll, 0.0)
    valid_ref[...] = valid.astype(jnp.float32)


def _masked_loss(x, t_col, first_row, rows_per_blk):
    n, v = x.shape
    grid = n // rows_per_blk
    return pl.pallas_call(
        functools.partial(_loss_body, rows_per_blk=rows_per_blk),
        grid=(grid,),
        in_specs=[
            pl.BlockSpec((rows_per_blk, v), lambda i: (i, 0)),
            pl.BlockSpec((rows_per_blk, 1), lambda i: (i, 0)),
            pl.BlockSpec((1, v), lambda i: (0, 0)),
        ],
        out_specs=[
            pl.BlockSpec((rows_per_blk, 1), lambda i: (i, 0)),
            pl.BlockSpec((rows_per_blk, 1), lambda i: (i, 0)),
        ],
        out_shape=[
            jax.ShapeDtypeStruct((n, 1), jnp.float32),
            jax.ShapeDtypeStruct((n, 1), jnp.float32),
        ],
    )(x, t_col, first_row)


def kernel(input, target):
    n = input.shape[-2] * input.shape[0]
    v = input.shape[-1]
    x = input.reshape(n, v)
    t = target.reshape(n).astype(jnp.int32)
    first = _first_occurrence(t, n, v)
    loss, valid = _masked_loss(x, t.reshape(n, 1), first.reshape(1, v), 128)
    return loss.sum() / valid.sum()
